# baseline (speedup 1.0000x reference)
import functools

import jax
import jax.numpy as jnp
from jax import lax
from jax.experimental import pallas as pl
from jax.experimental.pallas import tpu as pltpu
from jax.experimental.pallas import tpu_sc as plsc

N_NODES = 10000
FEATS = 128
LANES = 16
NCORES = 2
NSUB = 16
NWORKERS = NCORES * NSUB
CH = 80
GROUPS = CH // LANES
ZROWS = 624
REM = N_NODES - NSUB * ZROWS


def _matmul_body(p_ref, w_ref, b_ref, o_ref):
    x = p_ref[0] + p_ref[1]
    o_ref[...] = jnp.dot(x, w_ref[...],
                         preferred_element_type=jnp.float32) + b_ref[...]


def _sc_scatter(x, src, dst, ew):
    e_total = src.shape[0]
    per_worker = e_total // NWORKERS
    n_chunks = per_worker // CH

    mesh = plsc.VectorSubcoreMesh(core_axis_name="c", subcore_axis_name="s")

    @functools.partial(
        pl.kernel,
        mesh=mesh,
        out_type=jax.ShapeDtypeStruct((NCORES, N_NODES, FEATS), jnp.float32),
        scratch_types=[
            pltpu.VMEM((per_worker,), jnp.int32),
            pltpu.VMEM((per_worker,), jnp.int32),
            pltpu.VMEM((per_worker,), jnp.float32),
            pltpu.VMEM((CH, FEATS), jnp.float32),
            pltpu.VMEM((CH, FEATS), jnp.float32),
            pltpu.VMEM_SHARED((N_NODES, FEATS), jnp.float32),
            pltpu.SemaphoreType.DMA,
            pltpu.SemaphoreType.DMA,
        ],
    )
    def k(x_hbm, src_hbm, dst_hbm, ew_hbm, out_hbm,
          sidx_all, didx_all, w_all, rows0, rows1, acc, sem0, sem1):
        cid = lax.axis_index("c")
        sid = lax.axis_index("s")
        wid = cid * NSUB + sid
        base = pl.multiple_of(wid * per_worker, 8)

        pltpu.sync_copy(src_hbm.at[pl.ds(base, per_worker)], sidx_all)
        pltpu.sync_copy(dst_hbm.at[pl.ds(base, per_worker)], didx_all)
        pltpu.sync_copy(ew_hbm.at[pl.ds(base, per_worker)], w_all)

        NSPLIT = 5

        def gather_start(ck, buf, sem):
            eoff = pl.multiple_of(ck * CH, 16)
            for s in range(NSPLIT):
                sub = CH // NSPLIT
                idx = sidx_all.at[pl.ds(eoff + s * sub, sub)]
                pltpu.async_copy(x_hbm.at[idx],
                                 buf.at[pl.ds(s * sub, sub)], sem)

        def gather_wait(ck, buf, sem):
            eoff = pl.multiple_of(ck * CH, 16)
            for s in range(NSPLIT):
                sub = CH // NSPLIT
                idx = sidx_all.at[pl.ds(eoff + s * sub, sub)]
                pltpu.make_async_copy(x_hbm.at[idx],
                                      buf.at[pl.ds(s * sub, sub)],
                                      sem).wait()

        def scale(ck, buf):
            eoff = pl.multiple_of(ck * CH, 16)
            for g in range(GROUPS):
                goff = pl.multiple_of(eoff + g * LANES, 16)
                wg = w_all[pl.ds(goff, LANES)]
                for l in range(LANES):
                    wl = wg[l]
                    e = g * LANES + l
                    for j in range(FEATS // LANES):
                        sl = pl.ds(j * LANES, LANES)
                        buf[e, sl] = buf[e, sl] * wl

        def scatter(ck, buf):
            eoff = pl.multiple_of(ck * CH, 16)
            for g in range(GROUPS):
                goff = pl.multiple_of(eoff + g * LANES, 16)
                didx_g = didx_all[pl.ds(goff, LANES)]
                pltpu.sync_copy(buf.at[pl.ds(g * LANES, LANES)],
                                acc.at[didx_g], add=True)

        n_steady = (n_chunks - 3) // 2
        gather_start(0, rows0, sem0)

        zero16 = jnp.zeros((LANES,), jnp.float32)

        def zbody(e, c):
            for j in range(FEATS // LANES):
                rows1[e, pl.ds(j * LANES, LANES)] = zero16
            return c

        lax.fori_loop(0, CH, zbody, 0)

        zbase = sid * ZROWS
        off = 0
        while off < ZROWS:
            n = min(CH, ZROWS - off)
            pltpu.sync_copy(rows1.at[pl.ds(0, n)],
                            acc.at[pl.ds(zbase + off, n)])
            off += n

        @pl.when(sid == NSUB - 1)
        def _():
            pltpu.sync_copy(rows1.at[pl.ds(0, REM)],
                            acc.at[pl.ds(NSUB * ZROWS, REM)])

        plsc.subcore_barrier()
        gather_start(1, rows1, sem1)

        def pair_body(i, c):
            k0 = 2 * i
            gather_wait(k0, rows0, sem0)
            scale(k0, rows0)
            scatter(k0, rows0)
            gather_start(k0 + 2, rows0, sem0)
            gather_wait(k0 + 1, rows1, sem1)
            scale(k0 + 1, rows1)
            scatter(k0 + 1, rows1)
            gather_start(k0 + 3, rows1, sem1)
            return c

        lax.fori_loop(0, n_steady, pair_body, 0)

        k_a, k_b, k_c = n_chunks - 3, n_chunks - 2, n_chunks - 1
        gather_wait(k_a, rows0, sem0)
        scale(k_a, rows0)
        scatter(k_a, rows0)
        gather_start(k_c, rows0, sem0)
        gather_wait(k_b, rows1, sem1)
        scale(k_b, rows1)
        scatter(k_b, rows1)
        gather_wait(k_c, rows0, sem0)
        scale(k_c, rows0)
        scatter(k_c, rows0)

        plsc.subcore_barrier()

        pltpu.sync_copy(acc.at[pl.ds(zbase, ZROWS)],
                        out_hbm.at[cid, pl.ds(zbase, ZROWS)])

        @pl.when(sid == NSUB - 1)
        def _():
            pltpu.sync_copy(acc.at[pl.ds(NSUB * ZROWS, REM)],
                            out_hbm.at[cid, pl.ds(NSUB * ZROWS, REM)])

    return k(x, src, dst, ew)


def kernel(inputs, edge_index, edge_weight, W, B):
    n, in_feats = inputs.shape
    out_feats = W.shape[1]

    partials = _sc_scatter(inputs, edge_index[1], edge_index[0], edge_weight)

    out = pl.pallas_call(
        _matmul_body,
        grid=(5,),
        in_specs=[
            pl.BlockSpec((NCORES, n // 5, in_feats), lambda i: (0, i, 0)),
            pl.BlockSpec((in_feats, out_feats), lambda i: (0, 0)),
            pl.BlockSpec((1, out_feats), lambda i: (0, 0)),
        ],
        out_specs=pl.BlockSpec((n // 5, out_feats), lambda i: (i, 0)),
        out_shape=jax.ShapeDtypeStruct((n, out_feats), jnp.float32),
    )(partials, W, B.reshape(1, out_feats))

    return out


# speedup vs baseline: 1.3064x; 1.3064x over previous
import functools

import jax
import jax.numpy as jnp
from jax import lax
from jax.experimental import pallas as pl
from jax.experimental.pallas import tpu as pltpu
from jax.experimental.pallas import tpu_sc as plsc

N_NODES = 10000
FEATS = 128
LANES = 16
NCORES = 2
NSUB = 16
NWORKERS = NCORES * NSUB
CH = 80
GROUPS = CH // LANES
ZROWS = 624
REM = N_NODES - NSUB * ZROWS


def _matmul_body(p_ref, w_ref, b_ref, o_ref):
    x = p_ref[0] + p_ref[1]
    o_ref[...] = jnp.dot(x, w_ref[...],
                         preferred_element_type=jnp.float32) + b_ref[...]


def _sc_scatter(x, src, dst, ew):
    e_total = src.shape[0]
    per_worker = e_total // NWORKERS
    n_chunks = per_worker // CH

    mesh = plsc.VectorSubcoreMesh(core_axis_name="c", subcore_axis_name="s")

    @functools.partial(
        pl.kernel,
        mesh=mesh,
        out_type=jax.ShapeDtypeStruct((NCORES, N_NODES, FEATS), jnp.float32),
        scratch_types=[
            pltpu.VMEM((per_worker,), jnp.int32),
            pltpu.VMEM((per_worker,), jnp.int32),
            pltpu.VMEM((per_worker,), jnp.float32),
            pltpu.VMEM((CH, FEATS), jnp.float32),
            pltpu.VMEM((CH, FEATS), jnp.float32),
            pltpu.VMEM_SHARED((N_NODES, FEATS), jnp.float32),
            pltpu.SemaphoreType.DMA,
            pltpu.SemaphoreType.DMA,
        ],
    )
    def k(x_hbm, src_hbm, dst_hbm, ew_hbm, out_hbm,
          sidx_all, didx_all, w_all, rows0, rows1, acc, sem0, sem1):
        cid = lax.axis_index("c")
        sid = lax.axis_index("s")
        wid = cid * NSUB + sid
        base = pl.multiple_of(wid * per_worker, 8)

        pltpu.sync_copy(src_hbm.at[pl.ds(base, per_worker)], sidx_all)
        pltpu.sync_copy(dst_hbm.at[pl.ds(base, per_worker)], didx_all)
        pltpu.sync_copy(ew_hbm.at[pl.ds(base, per_worker)], w_all)

        def gather_start(ck, buf, sem):
            eoff = pl.multiple_of(ck * CH, 16)
            idx = sidx_all.at[pl.ds(eoff, CH)]
            return pltpu.async_copy(x_hbm.at[idx], buf, sem)

        def gather_wait(ck, buf, sem):
            eoff = pl.multiple_of(ck * CH, 16)
            idx = sidx_all.at[pl.ds(eoff, CH)]
            pltpu.make_async_copy(x_hbm.at[idx], buf, sem).wait()

        def scale(ck, buf):
            eoff = pl.multiple_of(ck * CH, 16)
            for g in range(GROUPS):
                goff = pl.multiple_of(eoff + g * LANES, 16)
                wg = w_all[pl.ds(goff, LANES)]
                for l in range(LANES):
                    wl = wg[l]
                    e = g * LANES + l
                    for j in range(FEATS // LANES):
                        sl = pl.ds(j * LANES, LANES)
                        buf[e, sl] = buf[e, sl] * wl

        def scatter(ck, buf):
            eoff = pl.multiple_of(ck * CH, 16)
            for g in range(GROUPS):
                goff = pl.multiple_of(eoff + g * LANES, 16)
                didx_g = didx_all[pl.ds(goff, LANES)]
                pltpu.sync_copy(buf.at[pl.ds(g * LANES, LANES)],
                                acc.at[didx_g], add=True)

        n_steady = (n_chunks - 3) // 2
        gather_start(0, rows0, sem0)

        zero16 = jnp.zeros((LANES,), jnp.float32)

        def zbody(e, c):
            for j in range(FEATS // LANES):
                rows1[e, pl.ds(j * LANES, LANES)] = zero16
            return c

        lax.fori_loop(0, CH, zbody, 0)

        zbase = sid * ZROWS
        off = 0
        while off < ZROWS:
            n = min(CH, ZROWS - off)
            pltpu.sync_copy(rows1.at[pl.ds(0, n)],
                            acc.at[pl.ds(zbase + off, n)])
            off += n

        @pl.when(sid == NSUB - 1)
        def _():
            pltpu.sync_copy(rows1.at[pl.ds(0, REM)],
                            acc.at[pl.ds(NSUB * ZROWS, REM)])

        plsc.subcore_barrier()
        gather_start(1, rows1, sem1)

        def pair_body(i, c):
            k0 = 2 * i
            gather_wait(k0, rows0, sem0)
            scale(k0, rows0)
            scatter(k0, rows0)
            gather_start(k0 + 2, rows0, sem0)
            gather_wait(k0 + 1, rows1, sem1)
            scale(k0 + 1, rows1)
            scatter(k0 + 1, rows1)
            gather_start(k0 + 3, rows1, sem1)
            return c

        lax.fori_loop(0, n_steady, pair_body, 0)

        k_a, k_b, k_c = n_chunks - 3, n_chunks - 2, n_chunks - 1
        gather_wait(k_a, rows0, sem0)
        scale(k_a, rows0)
        scatter(k_a, rows0)
        gather_start(k_c, rows0, sem0)
        gather_wait(k_b, rows1, sem1)
        scale(k_b, rows1)
        scatter(k_b, rows1)
        gather_wait(k_c, rows0, sem0)
        scale(k_c, rows0)
        scatter(k_c, rows0)

        plsc.subcore_barrier()

        pltpu.sync_copy(acc.at[pl.ds(zbase, ZROWS)],
                        out_hbm.at[cid, pl.ds(zbase, ZROWS)])

        @pl.when(sid == NSUB - 1)
        def _():
            pltpu.sync_copy(acc.at[pl.ds(NSUB * ZROWS, REM)],
                            out_hbm.at[cid, pl.ds(NSUB * ZROWS, REM)])

    return k(x, src, dst, ew)


def kernel(inputs, edge_index, edge_weight, W, B):
    n, in_feats = inputs.shape
    out_feats = W.shape[1]

    partials = _sc_scatter(inputs, edge_index[1], edge_index[0], edge_weight)

    out = pl.pallas_call(
        _matmul_body,
        grid=(5,),
        in_specs=[
            pl.BlockSpec((NCORES, n // 5, in_feats), lambda i: (0, i, 0)),
            pl.BlockSpec((in_feats, out_feats), lambda i: (0, 0)),
            pl.BlockSpec((1, out_feats), lambda i: (0, 0)),
        ],
        out_specs=pl.BlockSpec((n // 5, out_feats), lambda i: (i, 0)),
        out_shape=jax.ShapeDtypeStruct((n, out_feats), jnp.float32),
    )(partials, W, B.reshape(1, out_feats))

    return out


# speedup vs baseline: 1.3945x; 1.0675x over previous
import functools

import jax
import jax.numpy as jnp
from jax import lax
from jax.experimental import pallas as pl
from jax.experimental.pallas import tpu as pltpu
from jax.experimental.pallas import tpu_sc as plsc

N_NODES = 10000
FEATS = 128
LANES = 16
NCORES = 2
NSUB = 16
NWORKERS = NCORES * NSUB
CH = 80
GROUPS = CH // LANES
ZROWS = 624
REM = N_NODES - NSUB * ZROWS


def _matmul_body(p_ref, w_ref, b_ref, o_ref):
    x = p_ref[0] + p_ref[1]
    o_ref[...] = jnp.dot(x, w_ref[...],
                         preferred_element_type=jnp.float32) + b_ref[...]


def _sc_scatter(x, src, dst, ew):
    e_total = src.shape[0]
    per_worker = e_total // NWORKERS
    n_chunks = per_worker // CH

    mesh = plsc.VectorSubcoreMesh(core_axis_name="c", subcore_axis_name="s")

    @functools.partial(
        pl.kernel,
        mesh=mesh,
        out_type=jax.ShapeDtypeStruct((NCORES, N_NODES, FEATS), jnp.float32),
        scratch_types=[
            pltpu.VMEM((per_worker,), jnp.int32),
            pltpu.VMEM((per_worker,), jnp.int32),
            pltpu.VMEM((per_worker,), jnp.float32),
            pltpu.VMEM((CH, FEATS), jnp.float32),
            pltpu.VMEM((CH, FEATS), jnp.float32),
            pltpu.VMEM_SHARED((N_NODES, FEATS), jnp.float32),
            pltpu.SemaphoreType.DMA,
            pltpu.SemaphoreType.DMA,
        ],
    )
    def k(x_hbm, src_hbm, dst_hbm, ew_hbm, out_hbm,
          sidx_all, didx_all, w_all, rows0, rows1, acc, sem0, sem1):
        cid = lax.axis_index("c")
        sid = lax.axis_index("s")
        wid = cid * NSUB + sid
        base = pl.multiple_of(wid * per_worker, 8)

        pltpu.sync_copy(src_hbm.at[pl.ds(base, per_worker)], sidx_all)
        pltpu.sync_copy(dst_hbm.at[pl.ds(base, per_worker)], didx_all)
        pltpu.sync_copy(ew_hbm.at[pl.ds(base, per_worker)], w_all)

        def gather_start(ck, buf, sem):
            eoff = pl.multiple_of(ck * CH, 16)
            idx = sidx_all.at[pl.ds(eoff, CH)]
            return pltpu.async_copy(x_hbm.at[idx], buf, sem)

        def gather_wait(ck, buf, sem):
            eoff = pl.multiple_of(ck * CH, 16)
            idx = sidx_all.at[pl.ds(eoff, CH)]
            pltpu.make_async_copy(x_hbm.at[idx], buf, sem).wait()

        def scale(ck, buf):
            eoff = pl.multiple_of(ck * CH, 16)
            for g in range(GROUPS):
                goff = pl.multiple_of(eoff + g * LANES, 16)
                wg = w_all[pl.ds(goff, LANES)]
                for l in range(LANES):
                    wl = wg[l]
                    e = g * LANES + l
                    for j in range(FEATS // LANES):
                        sl = pl.ds(j * LANES, LANES)
                        buf[e, sl] = buf[e, sl] * wl

        def scatter(ck, buf):
            eoff = pl.multiple_of(ck * CH, 16)
            idx = didx_all.at[pl.ds(eoff, CH)]
            pltpu.sync_copy(buf, acc.at[idx], add=True)

        n_steady = (n_chunks - 3) // 2
        gather_start(0, rows0, sem0)

        zero16 = jnp.zeros((LANES,), jnp.float32)

        def zbody(e, c):
            for j in range(FEATS // LANES):
                rows1[e, pl.ds(j * LANES, LANES)] = zero16
            return c

        lax.fori_loop(0, CH, zbody, 0)

        zbase = sid * ZROWS
        off = 0
        while off < ZROWS:
            n = min(CH, ZROWS - off)
            pltpu.sync_copy(rows1.at[pl.ds(0, n)],
                            acc.at[pl.ds(zbase + off, n)])
            off += n

        @pl.when(sid == NSUB - 1)
        def _():
            pltpu.sync_copy(rows1.at[pl.ds(0, REM)],
                            acc.at[pl.ds(NSUB * ZROWS, REM)])

        plsc.subcore_barrier()
        gather_start(1, rows1, sem1)

        def pair_body(i, c):
            k0 = 2 * i
            gather_wait(k0, rows0, sem0)
            scale(k0, rows0)
            scatter(k0, rows0)
            gather_start(k0 + 2, rows0, sem0)
            gather_wait(k0 + 1, rows1, sem1)
            scale(k0 + 1, rows1)
            scatter(k0 + 1, rows1)
            gather_start(k0 + 3, rows1, sem1)
            return c

        lax.fori_loop(0, n_steady, pair_body, 0)

        k_a, k_b, k_c = n_chunks - 3, n_chunks - 2, n_chunks - 1
        gather_wait(k_a, rows0, sem0)
        scale(k_a, rows0)
        scatter(k_a, rows0)
        gather_start(k_c, rows0, sem0)
        gather_wait(k_b, rows1, sem1)
        scale(k_b, rows1)
        scatter(k_b, rows1)
        gather_wait(k_c, rows0, sem0)
        scale(k_c, rows0)
        scatter(k_c, rows0)

        plsc.subcore_barrier()

        pltpu.sync_copy(acc.at[pl.ds(zbase, ZROWS)],
                        out_hbm.at[cid, pl.ds(zbase, ZROWS)])

        @pl.when(sid == NSUB - 1)
        def _():
            pltpu.sync_copy(acc.at[pl.ds(NSUB * ZROWS, REM)],
                            out_hbm.at[cid, pl.ds(NSUB * ZROWS, REM)])

    return k(x, src, dst, ew)


def kernel(inputs, edge_index, edge_weight, W, B):
    n, in_feats = inputs.shape
    out_feats = W.shape[1]

    partials = _sc_scatter(inputs, edge_index[1], edge_index[0], edge_weight)

    out = pl.pallas_call(
        _matmul_body,
        grid=(5,),
        in_specs=[
            pl.BlockSpec((NCORES, n // 5, in_feats), lambda i: (0, i, 0)),
            pl.BlockSpec((in_feats, out_feats), lambda i: (0, 0)),
            pl.BlockSpec((1, out_feats), lambda i: (0, 0)),
        ],
        out_specs=pl.BlockSpec((n // 5, out_feats), lambda i: (i, 0)),
        out_shape=jax.ShapeDtypeStruct((n, out_feats), jnp.float32),
    )(partials, W, B.reshape(1, out_feats))

    return out
